# Initial kernel scaffold; baseline (speedup 1.0000x reference)
import jax
import jax.numpy as jnp
from jax.experimental import pallas as pl


def kernel(x_o, x_c, edge_index_o, edge_index_c, batch_o, W1o, b1o, W2o, b2o, W1c, b1c, W2c, b2c, Wl1, bl1):
    raise NotImplementedError("write your pallas kernel here")


# speedup vs baseline: 7.0734x; 7.0734x over previous
import functools

import jax
import jax.numpy as jnp
from jax import lax
from jax.experimental import pallas as pl
from jax.experimental.pallas import tpu as pltpu
from jax.experimental.pallas import tpu_sc as plsc

N = 10000
E = 320000
D = 128
H = 128
G = 128

NC = 2
NS = 16
CH = 128
CPT = 80
DCPT = 160
EPAD = NC * NS * CPT * CH
NPAD = 10240
RPT = NPAD // NS
BLK = 1024
NBLK = NPAD // BLK

_mesh = functools.partial(
    plsc.VectorSubcoreMesh,
    core_axis_name="c", subcore_axis_name="s", num_cores=NC, num_subcores=NS,
)


def _deg_body(dsts, deg_out, spdeg, idxbuf, onesv, dumpv):
    c = lax.axis_index("c")
    s = lax.axis_index("s")
    for j in range(8):
        onesv[pl.ds(j * 16, 16)] = jnp.ones((16,), jnp.float32)

    def _zero(i, t):
        dumpv[pl.ds(i * 16, 16)] = jnp.zeros((16,), jnp.float32)
        return t

    lax.fori_loop(0, RPT // 16, _zero, 0)
    pltpu.sync_copy(dumpv, spdeg.at[pl.ds(s * RPT, RPT)])
    pltpu.sync_copy(dsts.at[c, s], idxbuf)
    plsc.subcore_barrier()

    def _chunk(j, t):
        pltpu.sync_copy(onesv, spdeg.at[idxbuf.at[j]], add=True)
        return t

    lax.fori_loop(0, DCPT, _chunk, 0)
    plsc.subcore_barrier()
    pltpu.sync_copy(spdeg.at[pl.ds(s * RPT, RPT)], dumpv)
    pltpu.sync_copy(dumpv, deg_out.at[c, pl.ds(s * RPT, RPT)])


def _degrees(dsts):
    return pl.kernel(
        _deg_body,
        out_type=jax.ShapeDtypeStruct((NC, NPAD), jnp.float32),
        mesh=_mesh(),
        scratch_types=[
            pltpu.VMEM_SHARED((NPAD,), jnp.float32),
            pltpu.VMEM((DCPT, CH), jnp.int32),
            pltpu.VMEM((CH,), jnp.float32),
            pltpu.VMEM((RPT,), jnp.float32),
        ],
    )(dsts)


def _spmm_body(y, src4, dst4, part, acc, srcbuf, dstbuf, rows, sem):
    c = lax.axis_index("c")
    s = lax.axis_index("s")

    def _zero(i, t):
        for j in range(8):
            rows[i, pl.ds(j * 16, 16)] = jnp.zeros((16,), jnp.float32)
        return t

    lax.fori_loop(0, CH, _zero, 0)
    for k in range(RPT // CH):
        pltpu.sync_copy(rows, acc.at[pl.ds(s * RPT + k * CH, CH)])
    pltpu.sync_copy(src4.at[c, s], srcbuf)
    pltpu.sync_copy(dst4.at[c, s], dstbuf)
    plsc.subcore_barrier()

    def _chunk(j, t):
        pltpu.async_copy(y.at[srcbuf.at[j]], rows, sem).wait()
        pltpu.sync_copy(rows, acc.at[dstbuf.at[j]], add=True)
        return t

    lax.fori_loop(0, CPT, _chunk, 0)
    plsc.subcore_barrier()
    for k in range(RPT // CH):
        pltpu.sync_copy(acc.at[pl.ds(s * RPT + k * CH, CH)], rows)
        pltpu.sync_copy(rows, part.at[c, pl.ds(s * RPT + k * CH, CH)])


def _spmm(y, src4, dst4):
    return pl.kernel(
        _spmm_body,
        out_type=jax.ShapeDtypeStruct((NC, NPAD, H), jnp.float32),
        mesh=_mesh(),
        scratch_types=[
            pltpu.VMEM_SHARED((NPAD, H), jnp.float32),
            pltpu.VMEM((CPT, CH), jnp.int32),
            pltpu.VMEM((CPT, CH), jnp.int32),
            pltpu.VMEM((CH, H), jnp.float32),
            pltpu.SemaphoreType.DMA,
        ],
    )(y, src4, dst4)


def _mm_scale_body(x_ref, deg_ref, w_ref, o_ref):
    dis = lax.rsqrt(deg_ref[...] + 1.0)
    o_ref[...] = jnp.dot(
        x_ref[...], w_ref[...], preferred_element_type=jnp.float32) * dis


def _mm_scale(x, deg, w):
    return pl.pallas_call(
        _mm_scale_body,
        grid=(NBLK,),
        in_specs=[
            pl.BlockSpec((BLK, D), lambda i: (i, 0)),
            pl.BlockSpec((BLK, 1), lambda i: (i, 0)),
            pl.BlockSpec((D, H), lambda i: (0, 0)),
        ],
        out_specs=pl.BlockSpec((BLK, H), lambda i: (i, 0)),
        out_shape=jax.ShapeDtypeStruct((NPAD, H), jnp.float32),
    )(x, deg, w)


def _combine_mm_body(p_ref, y_ref, deg_ref, b_ref, w_ref, o_ref):
    dis = lax.rsqrt(deg_ref[...] + 1.0)
    p = p_ref[...]
    h = dis * (p[0] + p[1] + y_ref[...]) + b_ref[...]
    h = jnp.maximum(h, 0.0)
    o_ref[...] = jnp.dot(
        h, w_ref[...], preferred_element_type=jnp.float32) * dis


def _combine_mm(part, y, deg, b, w):
    return pl.pallas_call(
        _combine_mm_body,
        grid=(NBLK,),
        in_specs=[
            pl.BlockSpec((NC, BLK, H), lambda i: (0, i, 0)),
            pl.BlockSpec((BLK, H), lambda i: (i, 0)),
            pl.BlockSpec((BLK, 1), lambda i: (i, 0)),
            pl.BlockSpec((1, H), lambda i: (0, 0)),
            pl.BlockSpec((H, H), lambda i: (0, 0)),
        ],
        out_specs=pl.BlockSpec((BLK, H), lambda i: (i, 0)),
        out_shape=jax.ShapeDtypeStruct((NPAD, H), jnp.float32),
    )(part, y, deg, b, w)


def _final_body(po_ref, yo_ref, dego_ref, bo_ref, pc_ref, yc_ref, degc_ref,
                bc_ref, batch_ref, wl_ref, bl_ref,
                xo_ref, xc_ref, hout_ref, pool_acc):
    i = pl.program_id(0)
    diso = lax.rsqrt(dego_ref[...] + 1.0)
    po = po_ref[...]
    xo_ref[...] = diso * (po[0] + po[1] + yo_ref[...]) + bo_ref[...]
    disc = lax.rsqrt(degc_ref[...] + 1.0)
    pc = pc_ref[...]
    xc = disc * (pc[0] + pc[1] + yc_ref[...]) + bc_ref[...]
    xc_ref[...] = xc
    gids = lax.broadcasted_iota(jnp.int32, (BLK, G), 1)
    onehot = jnp.where(batch_ref[...] == gids, 1.0, 0.0)
    contrib = lax.dot_general(
        onehot, xc, (((0,), (0,)), ((), ())),
        preferred_element_type=jnp.float32)

    @pl.when(i == 0)
    def _():
        pool_acc[...] = contrib

    @pl.when(i > 0)
    def _():
        pool_acc[...] += contrib

    @pl.when(i == NBLK - 1)
    def _():
        hout_ref[...] = jnp.dot(
            pool_acc[...], wl_ref[...],
            preferred_element_type=jnp.float32) + bl_ref[...]


def _final(po, yo, dego, bo, pc, yc, degc, bc, batch, wl, bl):
    return pl.pallas_call(
        _final_body,
        grid=(NBLK,),
        in_specs=[
            pl.BlockSpec((NC, BLK, H), lambda i: (0, i, 0)),
            pl.BlockSpec((BLK, H), lambda i: (i, 0)),
            pl.BlockSpec((BLK, 1), lambda i: (i, 0)),
            pl.BlockSpec((1, H), lambda i: (0, 0)),
            pl.BlockSpec((NC, BLK, H), lambda i: (0, i, 0)),
            pl.BlockSpec((BLK, H), lambda i: (i, 0)),
            pl.BlockSpec((BLK, 1), lambda i: (i, 0)),
            pl.BlockSpec((1, H), lambda i: (0, 0)),
            pl.BlockSpec((BLK, 1), lambda i: (i, 0)),
            pl.BlockSpec((H, H), lambda i: (0, 0)),
            pl.BlockSpec((1, H), lambda i: (0, 0)),
        ],
        out_specs=[
            pl.BlockSpec((BLK, H), lambda i: (i, 0)),
            pl.BlockSpec((BLK, H), lambda i: (i, 0)),
            pl.BlockSpec((G, H), lambda i: (0, 0)),
        ],
        out_shape=[
            jax.ShapeDtypeStruct((NPAD, H), jnp.float32),
            jax.ShapeDtypeStruct((NPAD, H), jnp.float32),
            jax.ShapeDtypeStruct((G, H), jnp.float32),
        ],
        scratch_shapes=[pltpu.VMEM((G, H), jnp.float32)],
    )(po, yo, dego, bo, pc, yc, degc, bc, batch, wl, bl)


def _prep_edges(edge_index):
    src = edge_index[0].astype(jnp.int32)
    dst = edge_index[1].astype(jnp.int32)
    pad = EPAD - E
    src = jnp.concatenate([src, jnp.zeros((pad,), jnp.int32)])
    dst = jnp.concatenate([dst, jnp.full((pad,), N, jnp.int32)])
    src4 = src.reshape(NC, NS, CPT, CH)
    dst4 = dst.reshape(NC, NS, CPT, CH)
    dst_deg = dst.reshape(NS, DCPT, CH)
    return src4, dst4, dst_deg


def _pad_rows(x):
    return jnp.concatenate(
        [x.astype(jnp.float32), jnp.zeros((NPAD - N, x.shape[1]), jnp.float32)])


def kernel(x_o, x_c, edge_index_o, edge_index_c, batch_o,
           W1o, b1o, W2o, b2o, W1c, b1c, W2c, b2c, Wl1, bl1):
    src4_o, dst4_o, dstdeg_o = _prep_edges(edge_index_o)
    src4_c, dst4_c, dstdeg_c = _prep_edges(edge_index_c)
    deg2 = _degrees(jnp.stack([dstdeg_o, dstdeg_c]))
    deg_o = deg2[0].reshape(NPAD, 1)
    deg_c = deg2[1].reshape(NPAD, 1)

    xo_p = _pad_rows(x_o)
    xc_p = _pad_rows(x_c)
    batch_p = jnp.concatenate(
        [batch_o.astype(jnp.int32),
         jnp.full((NPAD - N,), -1, jnp.int32)]).reshape(NPAD, 1)

    y1o = _mm_scale(xo_p, deg_o, W1o)
    y1c = _mm_scale(xc_p, deg_c, W1c)
    p1o = _spmm(y1o, src4_o, dst4_o)
    p1c = _spmm(y1c, src4_c, dst4_c)
    y2o = _combine_mm(p1o, y1o, deg_o, b1o.reshape(1, H), W2o)
    y2c = _combine_mm(p1c, y1c, deg_c, b1c.reshape(1, H), W2c)
    p2o = _spmm(y2o, src4_o, dst4_o)
    p2c = _spmm(y2c, src4_c, dst4_c)
    xo2, xc2, h_out = _final(
        p2o, y2o, deg_o, b2o.reshape(1, H),
        p2c, y2c, deg_c, b2c.reshape(1, H),
        batch_p, Wl1, bl1.reshape(1, H))
    return (h_out, xo2[:N], xc2[:N])


# speedup vs baseline: 25.8325x; 3.6520x over previous
import functools

import jax
import jax.numpy as jnp
from jax import lax
from jax.experimental import pallas as pl
from jax.experimental.pallas import tpu as pltpu
from jax.experimental.pallas import tpu_sc as plsc

N = 10000
E = 320000
D = 128
H = 128
G = 128

NC = 2
NS = 16
CH = 80
CPT = 125
NBUF = 5
DCPT = 250
NPAD = 10240
RPT = NPAD // NS
BLK = 1024
NBLK = NPAD // BLK

_mesh = functools.partial(
    plsc.VectorSubcoreMesh,
    core_axis_name="c", subcore_axis_name="s", num_cores=NC, num_subcores=NS,
)


def _deg_body(dsts, deg_out, spdeg, idxbuf, onesv, dumpv):
    c = lax.axis_index("c")
    s = lax.axis_index("s")
    for j in range(CH // 16):
        onesv[pl.ds(j * 16, 16)] = jnp.ones((16,), jnp.float32)

    def _zero(i, t):
        dumpv[pl.ds(i * 16, 16)] = jnp.zeros((16,), jnp.float32)
        return t

    lax.fori_loop(0, RPT // 16, _zero, 0)
    pltpu.sync_copy(dumpv, spdeg.at[pl.ds(s * RPT, RPT)])
    pltpu.sync_copy(dsts.at[c, s], idxbuf)
    plsc.subcore_barrier()

    def _chunk(j, t):
        pltpu.sync_copy(onesv, spdeg.at[idxbuf.at[j]], add=True)
        return t

    lax.fori_loop(0, DCPT, _chunk, 0)
    plsc.subcore_barrier()
    pltpu.sync_copy(spdeg.at[pl.ds(s * RPT, RPT)], dumpv)
    pltpu.sync_copy(dumpv, deg_out.at[c, pl.ds(s * RPT, RPT)])


def _degrees(dsts):
    return pl.kernel(
        _deg_body,
        out_type=jax.ShapeDtypeStruct((NC, NPAD), jnp.float32),
        mesh=_mesh(),
        scratch_types=[
            pltpu.VMEM_SHARED((NPAD,), jnp.float32),
            pltpu.VMEM((DCPT, CH), jnp.int32),
            pltpu.VMEM((CH,), jnp.float32),
            pltpu.VMEM((RPT,), jnp.float32),
        ],
    )(dsts)


def _spmm_body(y, pk4, part, acc, pkbuf, srcb, dstb, rows, s0, s1):
    c = lax.axis_index("c")
    s = lax.axis_index("s")
    sems = (s0, s1)

    def _zero(i, t):
        for j in range(H // 16):
            rows[0, i, pl.ds(j * 16, 16)] = jnp.zeros((16,), jnp.float32)
        return t

    lax.fori_loop(0, CH, _zero, 0)
    for k in range(RPT // CH):
        pltpu.sync_copy(rows.at[0], acc.at[pl.ds(s * RPT + k * CH, CH)])
    pltpu.sync_copy(pk4.at[c, s], pkbuf)
    plsc.subcore_barrier()

    def _wait(b):
        pltpu.make_async_copy(y.at[pl.ds(0, CH)], rows.at[b], sems[b]).wait()

    def _gather(j, b):
        for q in range(CH // 16):
            v = pkbuf[j, pl.ds(q * 16, 16)]
            srcb[b, pl.ds(q * 16, 16)] = lax.bitwise_and(v, 16383)
            dstb[b, pl.ds(q * 16, 16)] = lax.shift_right_logical(v, 14)
        pltpu.async_copy(y.at[srcb.at[b]], rows.at[b], sems[b])

    def _scatter(j, b):
        pltpu.sync_copy(rows.at[b], acc.at[dstb.at[b]], add=True)

    _gather(0, 0)
    _gather(1, 1)

    def _step(t, u):
        base = t * 2
        _wait(0)
        _scatter(base, 0)
        _gather(base + 2, 0)
        _wait(1)
        _scatter(base + 1, 1)
        _gather(base + 3, 1)
        return u

    lax.fori_loop(0, (CPT - 3) // 2, _step, 0)
    _wait(0)
    _scatter(CPT - 3, 0)
    _gather(CPT - 1, 0)
    _wait(1)
    _scatter(CPT - 2, 1)
    _wait(0)
    _scatter(CPT - 1, 0)
    plsc.subcore_barrier()
    pltpu.sync_copy(acc.at[pl.ds(s * RPT, RPT)], part.at[c, pl.ds(s * RPT, RPT)])


def _spmm(y, pk4):
    return pl.kernel(
        _spmm_body,
        out_type=jax.ShapeDtypeStruct((NC, NPAD, H), jnp.float32),
        mesh=_mesh(),
        scratch_types=[
            pltpu.VMEM_SHARED((NPAD, H), jnp.float32),
            pltpu.VMEM((CPT, CH), jnp.int32),
            pltpu.VMEM((2, CH), jnp.int32),
            pltpu.VMEM((2, CH), jnp.int32),
            pltpu.VMEM((2, CH, H), jnp.float32),
            pltpu.SemaphoreType.DMA,
            pltpu.SemaphoreType.DMA,
        ],
    )(y, pk4)


def _mm_scale_body(x_ref, deg_ref, w_ref, o_ref):
    dis = lax.rsqrt(deg_ref[...] + 1.0)
    o_ref[...] = jnp.dot(
        x_ref[...], w_ref[...], preferred_element_type=jnp.float32) * dis


def _mm_scale(x, deg, w):
    return pl.pallas_call(
        _mm_scale_body,
        grid=(NBLK,),
        in_specs=[
            pl.BlockSpec((BLK, D), lambda i: (i, 0)),
            pl.BlockSpec((BLK, 1), lambda i: (i, 0)),
            pl.BlockSpec((D, H), lambda i: (0, 0)),
        ],
        out_specs=pl.BlockSpec((BLK, H), lambda i: (i, 0)),
        out_shape=jax.ShapeDtypeStruct((NPAD, H), jnp.float32),
    )(x, deg, w)


def _combine_mm_body(p_ref, y_ref, deg_ref, b_ref, w_ref, o_ref):
    dis = lax.rsqrt(deg_ref[...] + 1.0)
    p = p_ref[...]
    h = dis * (p[0] + p[1] + y_ref[...]) + b_ref[...]
    h = jnp.maximum(h, 0.0)
    o_ref[...] = jnp.dot(
        h, w_ref[...], preferred_element_type=jnp.float32) * dis


def _combine_mm(part, y, deg, b, w):
    return pl.pallas_call(
        _combine_mm_body,
        grid=(NBLK,),
        in_specs=[
            pl.BlockSpec((NC, BLK, H), lambda i: (0, i, 0)),
            pl.BlockSpec((BLK, H), lambda i: (i, 0)),
            pl.BlockSpec((BLK, 1), lambda i: (i, 0)),
            pl.BlockSpec((1, H), lambda i: (0, 0)),
            pl.BlockSpec((H, H), lambda i: (0, 0)),
        ],
        out_specs=pl.BlockSpec((BLK, H), lambda i: (i, 0)),
        out_shape=jax.ShapeDtypeStruct((NPAD, H), jnp.float32),
    )(part, y, deg, b, w)


def _final_body(po_ref, yo_ref, dego_ref, bo_ref, pc_ref, yc_ref, degc_ref,
                bc_ref, batch_ref, wl_ref, bl_ref,
                xo_ref, xc_ref, hout_ref, pool_acc):
    i = pl.program_id(0)
    diso = lax.rsqrt(dego_ref[...] + 1.0)
    po = po_ref[...]
    xo_ref[...] = diso * (po[0] + po[1] + yo_ref[...]) + bo_ref[...]
    disc = lax.rsqrt(degc_ref[...] + 1.0)
    pc = pc_ref[...]
    xc = disc * (pc[0] + pc[1] + yc_ref[...]) + bc_ref[...]
    xc_ref[...] = xc
    gids = lax.broadcasted_iota(jnp.int32, (BLK, G), 1)
    onehot = jnp.where(batch_ref[...] == gids, 1.0, 0.0)
    contrib = lax.dot_general(
        onehot, xc, (((0,), (0,)), ((), ())),
        preferred_element_type=jnp.float32)

    @pl.when(i == 0)
    def _():
        pool_acc[...] = contrib

    @pl.when(i > 0)
    def _():
        pool_acc[...] += contrib

    @pl.when(i == NBLK - 1)
    def _():
        hout_ref[...] = jnp.dot(
            pool_acc[...], wl_ref[...],
            preferred_element_type=jnp.float32) + bl_ref[...]


def _final(po, yo, dego, bo, pc, yc, degc, bc, batch, wl, bl):
    return pl.pallas_call(
        _final_body,
        grid=(NBLK,),
        in_specs=[
            pl.BlockSpec((NC, BLK, H), lambda i: (0, i, 0)),
            pl.BlockSpec((BLK, H), lambda i: (i, 0)),
            pl.BlockSpec((BLK, 1), lambda i: (i, 0)),
            pl.BlockSpec((1, H), lambda i: (0, 0)),
            pl.BlockSpec((NC, BLK, H), lambda i: (0, i, 0)),
            pl.BlockSpec((BLK, H), lambda i: (i, 0)),
            pl.BlockSpec((BLK, 1), lambda i: (i, 0)),
            pl.BlockSpec((1, H), lambda i: (0, 0)),
            pl.BlockSpec((BLK, 1), lambda i: (i, 0)),
            pl.BlockSpec((H, H), lambda i: (0, 0)),
            pl.BlockSpec((1, H), lambda i: (0, 0)),
        ],
        out_specs=[
            pl.BlockSpec((BLK, H), lambda i: (i, 0)),
            pl.BlockSpec((BLK, H), lambda i: (i, 0)),
            pl.BlockSpec((G, H), lambda i: (0, 0)),
        ],
        out_shape=[
            jax.ShapeDtypeStruct((NPAD, H), jnp.float32),
            jax.ShapeDtypeStruct((NPAD, H), jnp.float32),
            jax.ShapeDtypeStruct((G, H), jnp.float32),
        ],
        scratch_shapes=[pltpu.VMEM((G, H), jnp.float32)],
    )(po, yo, dego, bo, pc, yc, degc, bc, batch, wl, bl)


def _prep_edges(edge_index):
    src = edge_index[0].astype(jnp.int32)
    dst = edge_index[1].astype(jnp.int32)
    pk4 = (dst * 16384 + src).reshape(NC, NS, CPT, CH)
    dst_deg = dst.reshape(NS, DCPT, CH)
    return pk4, dst_deg


def _pad_rows(x):
    return jnp.concatenate(
        [x.astype(jnp.float32), jnp.zeros((NPAD - N, x.shape[1]), jnp.float32)])


def kernel(x_o, x_c, edge_index_o, edge_index_c, batch_o,
           W1o, b1o, W2o, b2o, W1c, b1c, W2c, b2c, Wl1, bl1):
    pk4_o, dstdeg_o = _prep_edges(edge_index_o)
    pk4_c, dstdeg_c = _prep_edges(edge_index_c)
    deg2 = _degrees(jnp.stack([dstdeg_o, dstdeg_c]))
    deg_o = deg2[0].reshape(NPAD, 1)
    deg_c = deg2[1].reshape(NPAD, 1)

    xo_p = _pad_rows(x_o)
    xc_p = _pad_rows(x_c)
    batch_p = jnp.concatenate(
        [batch_o.astype(jnp.int32),
         jnp.full((NPAD - N,), -1, jnp.int32)]).reshape(NPAD, 1)

    y1o = _mm_scale(xo_p, deg_o, W1o)
    y1c = _mm_scale(xc_p, deg_c, W1c)
    p1o = _spmm(y1o, pk4_o)
    p1c = _spmm(y1c, pk4_c)
    y2o = _combine_mm(p1o, y1o, deg_o, b1o.reshape(1, H), W2o)
    y2c = _combine_mm(p1c, y1c, deg_c, b1c.reshape(1, H), W2c)
    p2o = _spmm(y2o, pk4_o)
    p2c = _spmm(y2c, pk4_c)
    xo2, xc2, h_out = _final(
        p2o, y2o, deg_o, b2o.reshape(1, H),
        p2c, y2c, deg_c, b2c.reshape(1, H),
        batch_p, Wl1, bl1.reshape(1, H))
    return (h_out, xo2[:N], xc2[:N])


# speedup vs baseline: 26.6923x; 1.0333x over previous
import functools

import jax
import jax.numpy as jnp
from jax import lax
from jax.experimental import pallas as pl
from jax.experimental.pallas import tpu as pltpu
from jax.experimental.pallas import tpu_sc as plsc

N = 10000
E = 320000
D = 128
H = 128
G = 128

NC = 2
NS = 16
CH = 80
CPT = 250
DCPT = 250
NPAD = 10240
RPT = NPAD // NS
BLK = 1024
NBLK = NPAD // BLK

_mesh = functools.partial(
    plsc.VectorSubcoreMesh,
    core_axis_name="c", subcore_axis_name="s", num_cores=NC, num_subcores=NS,
)


def _deg_body(dsts, deg_out, spdeg, idxbuf, onesv, dumpv):
    c = lax.axis_index("c")
    s = lax.axis_index("s")
    for j in range(CH // 16):
        onesv[pl.ds(j * 16, 16)] = jnp.ones((16,), jnp.float32)

    def _zero(i, t):
        dumpv[pl.ds(i * 16, 16)] = jnp.zeros((16,), jnp.float32)
        return t

    lax.fori_loop(0, RPT // 16, _zero, 0)
    pltpu.sync_copy(dumpv, spdeg.at[pl.ds(s * RPT, RPT)])
    pltpu.sync_copy(dsts.at[c, s], idxbuf)
    plsc.subcore_barrier()

    def _chunk(j, t):
        pltpu.sync_copy(onesv, spdeg.at[idxbuf.at[j]], add=True)
        return t

    lax.fori_loop(0, DCPT, _chunk, 0)
    plsc.subcore_barrier()
    pltpu.sync_copy(spdeg.at[pl.ds(s * RPT, RPT)], dumpv)
    pltpu.sync_copy(dumpv, deg_out.at[c, pl.ds(s * RPT, RPT)])


def _degrees(dsts):
    return pl.kernel(
        _deg_body,
        out_type=jax.ShapeDtypeStruct((NC, NPAD), jnp.float32),
        mesh=_mesh(),
        scratch_types=[
            pltpu.VMEM_SHARED((NPAD,), jnp.float32),
            pltpu.VMEM((DCPT, CH), jnp.int32),
            pltpu.VMEM((CH,), jnp.float32),
            pltpu.VMEM((RPT,), jnp.float32),
        ],
    )(dsts)


def _spmm_body(ycat, pkall, part, acc, zbuf, pkchunk, srcb, dstb, rows,
               g0, g1, i0, i1):
    c = lax.axis_index("c")
    s = lax.axis_index("s")
    gsem = (g0, g1)
    isem = (i0, i1)

    def _idx_load(j, b):
        pltpu.async_copy(pkall.at[c, s, j], pkchunk.at[b], isem[b])

    def _idx_wait(b):
        pltpu.make_async_copy(
            pkall.at[c, s, 0], pkchunk.at[b], isem[b]).wait()

    def _unpack(b):
        for q in range(CH // 16):
            v = pkchunk[b, pl.ds(q * 16, 16)]
            srcb[b, pl.ds(q * 16, 16)] = lax.bitwise_and(v, 32767)
            dstb[b, pl.ds(q * 16, 16)] = lax.shift_right_logical(v, 15)

    def _gather(b):
        pltpu.async_copy(ycat.at[srcb.at[b]], rows.at[b], gsem[b])

    def _gather_wait(b):
        pltpu.make_async_copy(
            ycat.at[pl.ds(0, CH)], rows.at[b], gsem[b]).wait()

    def _scatter(b):
        pltpu.sync_copy(rows.at[b], acc.at[dstb.at[b]], add=True)

    _idx_load(0, 0)
    _idx_load(1, 1)

    def _zero(i, t):
        for j in range(H // 16):
            zbuf[i, pl.ds(j * 16, 16)] = jnp.zeros((16,), jnp.float32)
        return t

    lax.fori_loop(0, CH, _zero, 0)
    for b in range(2):
        _idx_wait(b)
        _unpack(b)
        _gather(b)
        _idx_load(2 + b, b)
    for k in range(RPT // CH):
        pltpu.sync_copy(zbuf, acc.at[pl.ds(s * RPT + k * CH, CH)])
    plsc.subcore_barrier()

    def _step(t, u):
        for b in range(2):
            _gather_wait(b)
            _scatter(b)
            _idx_wait(b)
            _unpack(b)
            _gather(b)
            _idx_load(2 * t + b + 4, b)
        return u

    lax.fori_loop(0, (CPT - 4) // 2, _step, 0)
    for b in range(2):
        _gather_wait(b)
        _scatter(b)
        _idx_wait(b)
        _unpack(b)
        _gather(b)
    for b in range(2):
        _gather_wait(b)
        _scatter(b)
    plsc.subcore_barrier()
    pltpu.sync_copy(acc.at[pl.ds(s * RPT, RPT)],
                    part.at[c, pl.ds(s * RPT, RPT)])


def _spmm(ycat, pkall):
    return pl.kernel(
        _spmm_body,
        out_type=jax.ShapeDtypeStruct((NC, NPAD, H), jnp.float32),
        mesh=_mesh(),
        scratch_types=[
            pltpu.VMEM_SHARED((NPAD, H), jnp.float32),
            pltpu.VMEM((CH, H), jnp.float32),
            pltpu.VMEM((2, CH), jnp.int32),
            pltpu.VMEM((2, CH), jnp.int32),
            pltpu.VMEM((2, CH), jnp.int32),
            pltpu.VMEM((2, CH, H), jnp.float32),
            pltpu.SemaphoreType.DMA,
            pltpu.SemaphoreType.DMA,
            pltpu.SemaphoreType.DMA,
            pltpu.SemaphoreType.DMA,
        ],
    )(ycat, pkall)


def _mm_scale_body(x_ref, deg_ref, w_ref, o_ref):
    dis = lax.rsqrt(deg_ref[...][0] + 1.0)
    o_ref[...] = (jnp.dot(
        x_ref[...][0], w_ref[...][0],
        preferred_element_type=jnp.float32) * dis)[None]


def _mm_scale(xs, degs, ws):
    return pl.pallas_call(
        _mm_scale_body,
        grid=(NC, NBLK),
        in_specs=[
            pl.BlockSpec((1, BLK, D), lambda b, i: (b, i, 0)),
            pl.BlockSpec((1, BLK, 1), lambda b, i: (b, i, 0)),
            pl.BlockSpec((1, D, H), lambda b, i: (b, 0, 0)),
        ],
        out_specs=pl.BlockSpec((1, BLK, H), lambda b, i: (b, i, 0)),
        out_shape=jax.ShapeDtypeStruct((NC, NPAD, H), jnp.float32),
    )(xs, degs, ws)


def _combine_mm_body(p_ref, y_ref, deg_ref, b_ref, w_ref, o_ref):
    dis = lax.rsqrt(deg_ref[...][0] + 1.0)
    h = dis * (p_ref[...][0] + y_ref[...][0]) + b_ref[...][0]
    h = jnp.maximum(h, 0.0)
    o_ref[...] = (jnp.dot(
        h, w_ref[...][0], preferred_element_type=jnp.float32) * dis)[None]


def _combine_mm(part, y, degs, bs, ws):
    return pl.pallas_call(
        _combine_mm_body,
        grid=(NC, NBLK),
        in_specs=[
            pl.BlockSpec((1, BLK, H), lambda b, i: (b, i, 0)),
            pl.BlockSpec((1, BLK, H), lambda b, i: (b, i, 0)),
            pl.BlockSpec((1, BLK, 1), lambda b, i: (b, i, 0)),
            pl.BlockSpec((1, 1, H), lambda b, i: (b, 0, 0)),
            pl.BlockSpec((1, H, H), lambda b, i: (b, 0, 0)),
        ],
        out_specs=pl.BlockSpec((1, BLK, H), lambda b, i: (b, i, 0)),
        out_shape=jax.ShapeDtypeStruct((NC, NPAD, H), jnp.float32),
    )(part, y, degs, bs, ws)


def _final_body(po_ref, yo_ref, dego_ref, bo_ref, pc_ref, yc_ref, degc_ref,
                bc_ref, batch_ref, wl_ref, bl_ref,
                xo_ref, xc_ref, hout_ref, pool_acc):
    i = pl.program_id(0)
    diso = lax.rsqrt(dego_ref[...][0] + 1.0)
    xo_ref[...] = diso * (po_ref[...][0] + yo_ref[...][0]) + bo_ref[...]
    disc = lax.rsqrt(degc_ref[...][0] + 1.0)
    xc = disc * (pc_ref[...][0] + yc_ref[...][0]) + bc_ref[...]
    xc_ref[...] = xc
    gids = lax.broadcasted_iota(jnp.int32, (BLK, G), 1)
    onehot = jnp.where(batch_ref[...] == gids, 1.0, 0.0)
    contrib = lax.dot_general(
        onehot, xc, (((0,), (0,)), ((), ())),
        preferred_element_type=jnp.float32)

    @pl.when(i == 0)
    def _():
        pool_acc[...] = contrib

    @pl.when(i > 0)
    def _():
        pool_acc[...] += contrib

    @pl.when(i == NBLK - 1)
    def _():
        hout_ref[...] = jnp.dot(
            pool_acc[...], wl_ref[...],
            preferred_element_type=jnp.float32) + bl_ref[...]


def _final(p2, y2, degs, bo, bc, batch, wl, bl):
    return pl.pallas_call(
        _final_body,
        grid=(NBLK,),
        in_specs=[
            pl.BlockSpec((1, BLK, H), lambda i: (0, i, 0)),
            pl.BlockSpec((1, BLK, H), lambda i: (0, i, 0)),
            pl.BlockSpec((1, BLK, 1), lambda i: (0, i, 0)),
            pl.BlockSpec((1, H), lambda i: (0, 0)),
            pl.BlockSpec((1, BLK, H), lambda i: (1, i, 0)),
            pl.BlockSpec((1, BLK, H), lambda i: (1, i, 0)),
            pl.BlockSpec((1, BLK, 1), lambda i: (1, i, 0)),
            pl.BlockSpec((1, H), lambda i: (0, 0)),
            pl.BlockSpec((BLK, 1), lambda i: (i, 0)),
            pl.BlockSpec((H, H), lambda i: (0, 0)),
            pl.BlockSpec((1, H), lambda i: (0, 0)),
        ],
        out_specs=[
            pl.BlockSpec((BLK, H), lambda i: (i, 0)),
            pl.BlockSpec((BLK, H), lambda i: (i, 0)),
            pl.BlockSpec((G, H), lambda i: (0, 0)),
        ],
        out_shape=[
            jax.ShapeDtypeStruct((NPAD, H), jnp.float32),
            jax.ShapeDtypeStruct((NPAD, H), jnp.float32),
            jax.ShapeDtypeStruct((G, H), jnp.float32),
        ],
        scratch_shapes=[pltpu.VMEM((G, H), jnp.float32)],
    )(p2, y2, degs, bo, p2, y2, degs, bc, batch, wl, bl)


def _prep_edges(edge_index, branch):
    src = edge_index[0].astype(jnp.int32) + branch * NPAD
    dst = edge_index[1].astype(jnp.int32)
    pk = (dst * 32768 + src).reshape(NS, CPT, CH)
    dst_deg = edge_index[1].astype(jnp.int32).reshape(NS, DCPT, CH)
    return pk, dst_deg


def _pad_rows(x):
    return jnp.concatenate(
        [x.astype(jnp.float32), jnp.zeros((NPAD - N, x.shape[1]), jnp.float32)])


def kernel(x_o, x_c, edge_index_o, edge_index_c, batch_o,
           W1o, b1o, W2o, b2o, W1c, b1c, W2c, b2c, Wl1, bl1):
    pk_o, dstdeg_o = _prep_edges(edge_index_o, 0)
    pk_c, dstdeg_c = _prep_edges(edge_index_c, 1)
    pkall = jnp.stack([pk_o, pk_c])
    deg2 = _degrees(jnp.stack([dstdeg_o, dstdeg_c]))
    degs = deg2.reshape(NC, NPAD, 1)

    xs = jnp.stack([_pad_rows(x_o), _pad_rows(x_c)])
    batch_p = jnp.concatenate(
        [batch_o.astype(jnp.int32),
         jnp.full((NPAD - N,), -1, jnp.int32)]).reshape(NPAD, 1)

    y1 = _mm_scale(xs, degs, jnp.stack([W1o, W1c]))
    p1 = _spmm(y1.reshape(NC * NPAD, H), pkall)
    y2 = _combine_mm(p1, y1, degs,
                     jnp.stack([b1o.reshape(1, H), b1c.reshape(1, H)]),
                     jnp.stack([W2o, W2c]))
    p2 = _spmm(y2.reshape(NC * NPAD, H), pkall)
    xo2, xc2, h_out = _final(
        p2, y2, degs, b2o.reshape(1, H), b2c.reshape(1, H),
        batch_p, Wl1, bl1.reshape(1, H))
    return (h_out, xo2[:N], xc2[:N])


# speedup vs baseline: 27.1132x; 1.0158x over previous
import functools

import jax
import jax.numpy as jnp
from jax import lax
from jax.experimental import pallas as pl
from jax.experimental.pallas import tpu as pltpu
from jax.experimental.pallas import tpu_sc as plsc

N = 10000
E = 320000
D = 128
H = 128
G = 128

NC = 2
NS = 16
CH = 80
CPT = 250
DCPT = 250
NPAD = 10240
DRPT = NPAD // NS
RPT = NPAD // NS
BLK = 1000
NBLK = N // BLK

_mesh = functools.partial(
    plsc.VectorSubcoreMesh,
    core_axis_name="c", subcore_axis_name="s", num_cores=NC, num_subcores=NS,
)


def _deg_body(dsts, deg_out, spdeg, idxbuf, onesv, dumpv):
    c = lax.axis_index("c")
    s = lax.axis_index("s")
    for j in range(CH // 16):
        onesv[pl.ds(j * 16, 16)] = jnp.ones((16,), jnp.float32)

    def _zero(i, t):
        dumpv[pl.ds(i * 16, 16)] = jnp.zeros((16,), jnp.float32)
        return t

    lax.fori_loop(0, DRPT // 16, _zero, 0)
    pltpu.sync_copy(dumpv, spdeg.at[pl.ds(s * DRPT, DRPT)])
    pltpu.sync_copy(dsts.at[c, s], idxbuf)
    plsc.subcore_barrier()

    def _chunk(j, t):
        pltpu.sync_copy(onesv, spdeg.at[idxbuf.at[j]], add=True)
        return t

    lax.fori_loop(0, DCPT, _chunk, 0)
    plsc.subcore_barrier()
    pltpu.sync_copy(spdeg.at[pl.ds(s * DRPT, DRPT)], dumpv)
    pltpu.sync_copy(dumpv, deg_out.at[c, pl.ds(s * DRPT, DRPT)])


def _degrees(dsts):
    return pl.kernel(
        _deg_body,
        out_type=jax.ShapeDtypeStruct((NC, NPAD), jnp.float32),
        mesh=_mesh(),
        scratch_types=[
            pltpu.VMEM_SHARED((NPAD,), jnp.float32),
            pltpu.VMEM((DCPT, CH), jnp.int32),
            pltpu.VMEM((CH,), jnp.float32),
            pltpu.VMEM((DRPT,), jnp.float32),
        ],
    )(dsts)


def _spmm_body(ycat, pkall, part, acc, zbuf, pkchunk, srcb, dstb, rows,
               g0, g1, i0, i1):
    c = lax.axis_index("c")
    s = lax.axis_index("s")
    gsem = (g0, g1)
    isem = (i0, i1)

    def _idx_load(j, b):
        pltpu.async_copy(pkall.at[c, s, j], pkchunk.at[b], isem[b])

    def _idx_wait(b):
        pltpu.make_async_copy(
            pkall.at[c, s, 0], pkchunk.at[b], isem[b]).wait()

    def _unpack(b):
        for q in range(CH // 16):
            v = pkchunk[b, pl.ds(q * 16, 16)]
            srcb[b, pl.ds(q * 16, 16)] = lax.bitwise_and(v, 32767)
            dstb[b, pl.ds(q * 16, 16)] = lax.shift_right_logical(v, 15)

    def _gather(b):
        pltpu.async_copy(ycat.at[srcb.at[b]], rows.at[b], gsem[b])

    def _gather_wait(b):
        pltpu.make_async_copy(
            ycat.at[pl.ds(0, CH)], rows.at[b], gsem[b]).wait()

    def _scatter(b):
        pltpu.sync_copy(rows.at[b], acc.at[dstb.at[b]], add=True)

    _idx_load(0, 0)
    _idx_load(1, 1)

    def _zero(i, t):
        for j in range(H // 16):
            zbuf[i, pl.ds(j * 16, 16)] = jnp.zeros((16,), jnp.float32)
        return t

    lax.fori_loop(0, CH, _zero, 0)
    for b in range(2):
        _idx_wait(b)
        _unpack(b)
        _gather(b)
        _idx_load(2 + b, b)
    for k in range(RPT // CH):
        pltpu.sync_copy(zbuf, acc.at[pl.ds(s * RPT + k * CH, CH)])
    plsc.subcore_barrier()

    def _step(t, u):
        for b in range(2):
            _gather_wait(b)
            _scatter(b)
            _idx_wait(b)
            _unpack(b)
            _gather(b)
            _idx_load(2 * t + b + 4, b)
        return u

    lax.fori_loop(0, (CPT - 4) // 2, _step, 0)
    for b in range(2):
        _gather_wait(b)
        _scatter(b)
        _idx_wait(b)
        _unpack(b)
        _gather(b)
    for b in range(2):
        _gather_wait(b)
        _scatter(b)
    plsc.subcore_barrier()
    pltpu.sync_copy(acc.at[pl.ds(s * RPT, RPT)],
                    part.at[c, pl.ds(s * RPT, RPT)])


def _spmm(ycat, pkall):
    return pl.kernel(
        _spmm_body,
        out_type=jax.ShapeDtypeStruct((NC, NPAD, H), jnp.float32),
        mesh=_mesh(),
        scratch_types=[
            pltpu.VMEM_SHARED((NPAD, H), jnp.float32),
            pltpu.VMEM((CH, H), jnp.float32),
            pltpu.VMEM((2, CH), jnp.int32),
            pltpu.VMEM((2, CH), jnp.int32),
            pltpu.VMEM((2, CH), jnp.int32),
            pltpu.VMEM((2, CH, H), jnp.float32),
            pltpu.SemaphoreType.DMA,
            pltpu.SemaphoreType.DMA,
            pltpu.SemaphoreType.DMA,
            pltpu.SemaphoreType.DMA,
        ],
    )(ycat, pkall)


def _mm_scale_body(x_ref, deg_ref, w_ref, o_ref):
    dis = lax.rsqrt(deg_ref[...][0] + 1.0)
    o_ref[...] = (jnp.dot(
        x_ref[...][0], w_ref[...][0],
        preferred_element_type=jnp.float32) * dis)[None]


def _mm_scale(xs, degs, ws):
    return pl.pallas_call(
        _mm_scale_body,
        grid=(NC, NBLK),
        in_specs=[
            pl.BlockSpec((1, BLK, D), lambda b, i: (b, i, 0)),
            pl.BlockSpec((1, BLK, 1), lambda b, i: (b, i, 0)),
            pl.BlockSpec((1, D, H), lambda b, i: (b, 0, 0)),
        ],
        out_specs=pl.BlockSpec((1, BLK, H), lambda b, i: (b, i, 0)),
        out_shape=jax.ShapeDtypeStruct((NC, N, H), jnp.float32),
    )(xs, degs, ws)


def _combine_mm_body(p_ref, y_ref, deg_ref, b_ref, w_ref, o_ref):
    dis = lax.rsqrt(deg_ref[...][0] + 1.0)
    h = dis * (p_ref[...][0] + y_ref[...][0]) + b_ref[...][0]
    h = jnp.maximum(h, 0.0)
    o_ref[...] = (jnp.dot(
        h, w_ref[...][0], preferred_element_type=jnp.float32) * dis)[None]


def _combine_mm(part, y, degs, bs, ws):
    return pl.pallas_call(
        _combine_mm_body,
        grid=(NC, NBLK),
        in_specs=[
            pl.BlockSpec((1, BLK, H), lambda b, i: (b, i, 0)),
            pl.BlockSpec((1, BLK, H), lambda b, i: (b, i, 0)),
            pl.BlockSpec((1, BLK, 1), lambda b, i: (b, i, 0)),
            pl.BlockSpec((1, 1, H), lambda b, i: (b, 0, 0)),
            pl.BlockSpec((1, H, H), lambda b, i: (b, 0, 0)),
        ],
        out_specs=pl.BlockSpec((1, BLK, H), lambda b, i: (b, i, 0)),
        out_shape=jax.ShapeDtypeStruct((NC, N, H), jnp.float32),
    )(part, y, degs, bs, ws)


def _final_body(po_ref, yo_ref, dego_ref, bo_ref, pc_ref, yc_ref, degc_ref,
                bc_ref, batch_ref, wl_ref, bl_ref,
                xo_ref, xc_ref, hout_ref, pool_acc):
    i = pl.program_id(0)
    diso = lax.rsqrt(dego_ref[...][0] + 1.0)
    xo_ref[...] = diso * (po_ref[...][0] + yo_ref[...][0]) + bo_ref[...]
    disc = lax.rsqrt(degc_ref[...][0] + 1.0)
    xc = disc * (pc_ref[...][0] + yc_ref[...][0]) + bc_ref[...]
    xc_ref[...] = xc
    gids = lax.broadcasted_iota(jnp.int32, (BLK, G), 1)
    onehot = jnp.where(batch_ref[...] == gids, 1.0, 0.0)
    contrib = lax.dot_general(
        onehot, xc, (((0,), (0,)), ((), ())),
        preferred_element_type=jnp.float32)

    @pl.when(i == 0)
    def _():
        pool_acc[...] = contrib

    @pl.when(i > 0)
    def _():
        pool_acc[...] += contrib

    @pl.when(i == NBLK - 1)
    def _():
        hout_ref[...] = jnp.dot(
            pool_acc[...], wl_ref[...],
            preferred_element_type=jnp.float32) + bl_ref[...]


def _final(p2, y2, degs, bo, bc, batch, wl, bl):
    return pl.pallas_call(
        _final_body,
        grid=(NBLK,),
        in_specs=[
            pl.BlockSpec((1, BLK, H), lambda i: (0, i, 0)),
            pl.BlockSpec((1, BLK, H), lambda i: (0, i, 0)),
            pl.BlockSpec((1, BLK, 1), lambda i: (0, i, 0)),
            pl.BlockSpec((1, H), lambda i: (0, 0)),
            pl.BlockSpec((1, BLK, H), lambda i: (1, i, 0)),
            pl.BlockSpec((1, BLK, H), lambda i: (1, i, 0)),
            pl.BlockSpec((1, BLK, 1), lambda i: (1, i, 0)),
            pl.BlockSpec((1, H), lambda i: (0, 0)),
            pl.BlockSpec((BLK, 1), lambda i: (i, 0)),
            pl.BlockSpec((H, H), lambda i: (0, 0)),
            pl.BlockSpec((1, H), lambda i: (0, 0)),
        ],
        out_specs=[
            pl.BlockSpec((BLK, H), lambda i: (i, 0)),
            pl.BlockSpec((BLK, H), lambda i: (i, 0)),
            pl.BlockSpec((G, H), lambda i: (0, 0)),
        ],
        out_shape=[
            jax.ShapeDtypeStruct((N, H), jnp.float32),
            jax.ShapeDtypeStruct((N, H), jnp.float32),
            jax.ShapeDtypeStruct((G, H), jnp.float32),
        ],
        scratch_shapes=[pltpu.VMEM((G, H), jnp.float32)],
    )(p2, y2, degs, bo, p2, y2, degs, bc, batch, wl, bl)


def _prep_edges(edge_index, branch):
    src = edge_index[0].astype(jnp.int32) + branch * N
    dst = edge_index[1].astype(jnp.int32)
    pk = (dst * 32768 + src).reshape(NS, CPT, CH)
    dst_deg = edge_index[1].astype(jnp.int32).reshape(NS, DCPT, CH)
    return pk, dst_deg


def kernel(x_o, x_c, edge_index_o, edge_index_c, batch_o,
           W1o, b1o, W2o, b2o, W1c, b1c, W2c, b2c, Wl1, bl1):
    pk_o, dstdeg_o = _prep_edges(edge_index_o, 0)
    pk_c, dstdeg_c = _prep_edges(edge_index_c, 1)
    pkall = jnp.stack([pk_o, pk_c])
    deg2 = _degrees(jnp.stack([dstdeg_o, dstdeg_c]))
    degs = deg2.reshape(NC, NPAD, 1)

    xs = jnp.stack([x_o.astype(jnp.float32), x_c.astype(jnp.float32)])
    batch_p = batch_o.astype(jnp.int32).reshape(N, 1)

    y1 = _mm_scale(xs, degs, jnp.stack([W1o, W1c]))
    p1 = _spmm(y1.reshape(NC * N, H), pkall)
    y2 = _combine_mm(p1, y1, degs,
                     jnp.stack([b1o.reshape(1, H), b1c.reshape(1, H)]),
                     jnp.stack([W2o, W2c]))
    p2 = _spmm(y2.reshape(NC * N, H), pkall)
    xo2, xc2, h_out = _final(
        p2, y2, degs, b2o.reshape(1, H), b2c.reshape(1, H),
        batch_p, Wl1, bl1.reshape(1, H))
    return (h_out, xo2, xc2)


# speedup vs baseline: 31.8461x; 1.1746x over previous
import functools

import jax
import jax.numpy as jnp
from jax import lax
from jax.experimental import pallas as pl
from jax.experimental.pallas import tpu as pltpu
from jax.experimental.pallas import tpu_sc as plsc

N = 10000
E = 320000
D = 128
H = 128
G = 128

NC = 2
NS = 16
CH = 80
CPT = 250
DCPT = 250
NPAD = 10240
DRPT = NPAD // NS
RPT = NPAD // NS
BLK = 1000
NBLK = N // BLK

_mesh = functools.partial(
    plsc.VectorSubcoreMesh,
    core_axis_name="c", subcore_axis_name="s", num_cores=NC, num_subcores=NS,
)


def _deg_body(dsts, deg_out, spdeg, idxbuf, onesv, dumpv):
    c = lax.axis_index("c")
    s = lax.axis_index("s")
    for j in range(CH // 16):
        onesv[pl.ds(j * 16, 16)] = jnp.ones((16,), jnp.float32)

    def _zero(i, t):
        dumpv[pl.ds(i * 16, 16)] = jnp.zeros((16,), jnp.float32)
        return t

    lax.fori_loop(0, DRPT // 16, _zero, 0)
    pltpu.sync_copy(dumpv, spdeg.at[pl.ds(s * DRPT, DRPT)])
    pltpu.sync_copy(dsts.at[c, s], idxbuf)
    plsc.subcore_barrier()

    def _chunk(j, t):
        pltpu.sync_copy(onesv, spdeg.at[idxbuf.at[j]], add=True)
        return t

    lax.fori_loop(0, DCPT, _chunk, 0)
    plsc.subcore_barrier()
    pltpu.sync_copy(spdeg.at[pl.ds(s * DRPT, DRPT)], dumpv)
    pltpu.sync_copy(dumpv, deg_out.at[c, pl.ds(s * DRPT, DRPT)])


def _degrees(dsts):
    return pl.kernel(
        _deg_body,
        out_type=jax.ShapeDtypeStruct((NC, NPAD), jnp.float32),
        mesh=_mesh(),
        scratch_types=[
            pltpu.VMEM_SHARED((NPAD,), jnp.float32),
            pltpu.VMEM((DCPT, CH), jnp.int32),
            pltpu.VMEM((CH,), jnp.float32),
            pltpu.VMEM((DRPT,), jnp.float32),
        ],
    )(dsts)


def _spmm_body(ycat, pkall, part, acc, pkchunk, srcb, dstb, rows,
               g0, g1, g2, i0, i1, i2):
    c = lax.axis_index("c")
    s = lax.axis_index("s")
    gsem = (g0, g1, g2)
    isem = (i0, i1, i2)

    def _idx_load(j, b):
        pltpu.async_copy(pkall.at[c, s, j], pkchunk.at[b], isem[b])

    def _idx_wait(b):
        pltpu.make_async_copy(
            pkall.at[c, s, 0], pkchunk.at[b], isem[b]).wait()

    def _unpack(b):
        for q in range(CH // 16):
            v = pkchunk[b, pl.ds(q * 16, 16)]
            srcb[b, pl.ds(q * 16, 16)] = lax.bitwise_and(v, 32767)
            dstb[b, pl.ds(q * 16, 16)] = lax.shift_right_logical(v, 15)

    def _gather(b):
        pltpu.async_copy(ycat.at[srcb.at[b]], rows.at[b], gsem[b])

    def _gather_wait(b):
        pltpu.make_async_copy(
            ycat.at[pl.ds(0, CH)], rows.at[b], gsem[b]).wait()

    def _scatter(b):
        pltpu.sync_copy(rows.at[b], acc.at[dstb.at[b]], add=True)

    for b in range(3):
        _idx_load(b, b)

    def _zero(i, t):
        for j in range(H // 16):
            rows[0, i, pl.ds(j * 16, 16)] = jnp.zeros((16,), jnp.float32)
        return t

    lax.fori_loop(0, CH, _zero, 0)
    for k in range(RPT // CH):
        pltpu.sync_copy(rows.at[0], acc.at[pl.ds(s * RPT + k * CH, CH)])
    for b in range(3):
        _idx_wait(b)
        _unpack(b)
        _gather(b)
        _idx_load(b + 3, b)
    plsc.subcore_barrier()

    def _step(t, u):
        for b in range(3):
            _gather_wait(b)
            _scatter(b)
            _idx_wait(b)
            _unpack(b)
            _gather(b)
            _idx_load(3 * t + b + 6, b)
        return u

    lax.fori_loop(0, (CPT - 7) // 3, _step, 0)
    for b in range(3):
        _gather_wait(b)
        _scatter(b)
        _idx_wait(b)
        _unpack(b)
        _gather(b)
    _idx_load(CPT - 1, 0)
    _gather_wait(0)
    _scatter(0)
    _idx_wait(0)
    _unpack(0)
    _gather(0)
    _gather_wait(1)
    _scatter(1)
    _gather_wait(2)
    _scatter(2)
    _gather_wait(0)
    _scatter(0)
    plsc.subcore_barrier()
    pltpu.sync_copy(acc.at[pl.ds(s * RPT, RPT)],
                    part.at[c, pl.ds(s * RPT, RPT)])


def _spmm(ycat, pkall):
    return pl.kernel(
        _spmm_body,
        out_type=jax.ShapeDtypeStruct((NC, NPAD, H), jnp.float32),
        mesh=_mesh(),
        scratch_types=[
            pltpu.VMEM_SHARED((NPAD, H), jnp.float32),
            pltpu.VMEM((3, CH), jnp.int32),
            pltpu.VMEM((3, CH), jnp.int32),
            pltpu.VMEM((3, CH), jnp.int32),
            pltpu.VMEM((3, CH, H), jnp.float32),
            pltpu.SemaphoreType.DMA,
            pltpu.SemaphoreType.DMA,
            pltpu.SemaphoreType.DMA,
            pltpu.SemaphoreType.DMA,
            pltpu.SemaphoreType.DMA,
            pltpu.SemaphoreType.DMA,
        ],
    )(ycat, pkall)


def _mm_scale_body(x_ref, deg_ref, w_ref, o_ref):
    dis = lax.rsqrt(deg_ref[...][0] + 1.0)
    o_ref[...] = (jnp.dot(
        x_ref[...][0], w_ref[...][0],
        preferred_element_type=jnp.float32) * dis)[None]


def _mm_scale(xs, degs, ws):
    return pl.pallas_call(
        _mm_scale_body,
        grid=(NC, NBLK),
        in_specs=[
            pl.BlockSpec((1, BLK, D), lambda b, i: (b, i, 0)),
            pl.BlockSpec((1, BLK, 1), lambda b, i: (b, i, 0)),
            pl.BlockSpec((1, D, H), lambda b, i: (b, 0, 0)),
        ],
        out_specs=pl.BlockSpec((1, BLK, H), lambda b, i: (b, i, 0)),
        out_shape=jax.ShapeDtypeStruct((NC, N, H), jnp.float32),
    )(xs, degs, ws)


def _combine_mm_body(p_ref, y_ref, deg_ref, b_ref, w_ref, o_ref):
    dis = lax.rsqrt(deg_ref[...][0] + 1.0)
    h = dis * (p_ref[...][0] + y_ref[...][0]) + b_ref[...][0]
    h = jnp.maximum(h, 0.0)
    o_ref[...] = (jnp.dot(
        h, w_ref[...][0], preferred_element_type=jnp.float32) * dis)[None]


def _combine_mm(part, y, degs, bs, ws):
    return pl.pallas_call(
        _combine_mm_body,
        grid=(NC, NBLK),
        in_specs=[
            pl.BlockSpec((1, BLK, H), lambda b, i: (b, i, 0)),
            pl.BlockSpec((1, BLK, H), lambda b, i: (b, i, 0)),
            pl.BlockSpec((1, BLK, 1), lambda b, i: (b, i, 0)),
            pl.BlockSpec((1, 1, H), lambda b, i: (b, 0, 0)),
            pl.BlockSpec((1, H, H), lambda b, i: (b, 0, 0)),
        ],
        out_specs=pl.BlockSpec((1, BLK, H), lambda b, i: (b, i, 0)),
        out_shape=jax.ShapeDtypeStruct((NC, N, H), jnp.float32),
    )(part, y, degs, bs, ws)


def _final_body(po_ref, yo_ref, dego_ref, bo_ref, pc_ref, yc_ref, degc_ref,
                bc_ref, batch_ref, wl_ref, bl_ref,
                xo_ref, xc_ref, hout_ref, pool_acc):
    i = pl.program_id(0)
    diso = lax.rsqrt(dego_ref[...][0] + 1.0)
    xo_ref[...] = diso * (po_ref[...][0] + yo_ref[...][0]) + bo_ref[...]
    disc = lax.rsqrt(degc_ref[...][0] + 1.0)
    xc = disc * (pc_ref[...][0] + yc_ref[...][0]) + bc_ref[...]
    xc_ref[...] = xc
    gids = lax.broadcasted_iota(jnp.int32, (BLK, G), 1)
    onehot = jnp.where(batch_ref[...] == gids, 1.0, 0.0)
    contrib = lax.dot_general(
        onehot, xc, (((0,), (0,)), ((), ())),
        preferred_element_type=jnp.float32)

    @pl.when(i == 0)
    def _():
        pool_acc[...] = contrib

    @pl.when(i > 0)
    def _():
        pool_acc[...] += contrib

    @pl.when(i == NBLK - 1)
    def _():
        hout_ref[...] = jnp.dot(
            pool_acc[...], wl_ref[...],
            preferred_element_type=jnp.float32) + bl_ref[...]


def _final(p2, y2, degs, bo, bc, batch, wl, bl):
    return pl.pallas_call(
        _final_body,
        grid=(NBLK,),
        in_specs=[
            pl.BlockSpec((1, BLK, H), lambda i: (0, i, 0)),
            pl.BlockSpec((1, BLK, H), lambda i: (0, i, 0)),
            pl.BlockSpec((1, BLK, 1), lambda i: (0, i, 0)),
            pl.BlockSpec((1, H), lambda i: (0, 0)),
            pl.BlockSpec((1, BLK, H), lambda i: (1, i, 0)),
            pl.BlockSpec((1, BLK, H), lambda i: (1, i, 0)),
            pl.BlockSpec((1, BLK, 1), lambda i: (1, i, 0)),
            pl.BlockSpec((1, H), lambda i: (0, 0)),
            pl.BlockSpec((BLK, 1), lambda i: (i, 0)),
            pl.BlockSpec((H, H), lambda i: (0, 0)),
            pl.BlockSpec((1, H), lambda i: (0, 0)),
        ],
        out_specs=[
            pl.BlockSpec((BLK, H), lambda i: (i, 0)),
            pl.BlockSpec((BLK, H), lambda i: (i, 0)),
            pl.BlockSpec((G, H), lambda i: (0, 0)),
        ],
        out_shape=[
            jax.ShapeDtypeStruct((N, H), jnp.float32),
            jax.ShapeDtypeStruct((N, H), jnp.float32),
            jax.ShapeDtypeStruct((G, H), jnp.float32),
        ],
        scratch_shapes=[pltpu.VMEM((G, H), jnp.float32)],
    )(p2, y2, degs, bo, p2, y2, degs, bc, batch, wl, bl)


def _prep_edges(edge_index, branch):
    src = edge_index[0].astype(jnp.int32) + branch * N
    dst = edge_index[1].astype(jnp.int32)
    pk = (dst * 32768 + src).reshape(NS, CPT, CH)
    dst_deg = edge_index[1].astype(jnp.int32).reshape(NS, DCPT, CH)
    return pk, dst_deg


def kernel(x_o, x_c, edge_index_o, edge_index_c, batch_o,
           W1o, b1o, W2o, b2o, W1c, b1c, W2c, b2c, Wl1, bl1):
    pk_o, dstdeg_o = _prep_edges(edge_index_o, 0)
    pk_c, dstdeg_c = _prep_edges(edge_index_c, 1)
    pkall = jnp.stack([pk_o, pk_c])
    deg2 = _degrees(jnp.stack([dstdeg_o, dstdeg_c]))
    degs = deg2.reshape(NC, NPAD, 1)

    xs = jnp.stack([x_o.astype(jnp.float32), x_c.astype(jnp.float32)])
    batch_p = batch_o.astype(jnp.int32).reshape(N, 1)

    y1 = _mm_scale(xs, degs, jnp.stack([W1o, W1c]))
    p1 = _spmm(y1.reshape(NC * N, H), pkall)
    y2 = _combine_mm(p1, y1, degs,
                     jnp.stack([b1o.reshape(1, H), b1c.reshape(1, H)]),
                     jnp.stack([W2o, W2c]))
    p2 = _spmm(y2.reshape(NC * N, H), pkall)
    xo2, xc2, h_out = _final(
        p2, y2, degs, b2o.reshape(1, H), b2c.reshape(1, H),
        batch_p, Wl1, bl1.reshape(1, H))
    return (h_out, xo2, xc2)


# speedup vs baseline: 32.1848x; 1.0106x over previous
import functools

import jax
import jax.numpy as jnp
from jax import lax
from jax.experimental import pallas as pl
from jax.experimental.pallas import tpu as pltpu
from jax.experimental.pallas import tpu_sc as plsc

N = 10000
E = 320000
D = 128
H = 128
G = 128

NC = 2
NS = 16
CH = 80
CPT = 250
DCPT = 250
NPAD = 10240
DRPT = NPAD // NS
RPT = NPAD // NS
BLK = 1000
NBLK = N // BLK

_mesh = functools.partial(
    plsc.VectorSubcoreMesh,
    core_axis_name="c", subcore_axis_name="s", num_cores=NC, num_subcores=NS,
)


def _deg_body(pkall, deg_out, spdeg, idxbuf, dstbuf, onesv, dumpv):
    c = lax.axis_index("c")
    s = lax.axis_index("s")
    for j in range(CH // 16):
        onesv[pl.ds(j * 16, 16)] = jnp.ones((16,), jnp.float32)

    def _zero(i, t):
        dumpv[pl.ds(i * 16, 16)] = jnp.zeros((16,), jnp.float32)
        return t

    lax.fori_loop(0, DRPT // 16, _zero, 0)
    pltpu.sync_copy(dumpv, spdeg.at[pl.ds(s * DRPT, DRPT)])
    pltpu.sync_copy(pkall.at[c, s], idxbuf)

    def _unp(j, t):
        for q in range(CH // 16):
            dstbuf[j, pl.ds(q * 16, 16)] = lax.shift_right_logical(
                idxbuf[j, pl.ds(q * 16, 16)], 15)
        return t

    lax.fori_loop(0, DCPT, _unp, 0)
    plsc.subcore_barrier()

    def _chunk(j, t):
        pltpu.sync_copy(onesv, spdeg.at[dstbuf.at[j]], add=True)
        return t

    lax.fori_loop(0, DCPT, _chunk, 0)
    plsc.subcore_barrier()
    pltpu.sync_copy(spdeg.at[pl.ds(s * DRPT, DRPT)], dumpv)
    pltpu.sync_copy(dumpv, deg_out.at[c, pl.ds(s * DRPT, DRPT)])


def _degrees(pkall):
    return pl.kernel(
        _deg_body,
        out_type=jax.ShapeDtypeStruct((NC, NPAD), jnp.float32),
        mesh=_mesh(),
        scratch_types=[
            pltpu.VMEM_SHARED((NPAD,), jnp.float32),
            pltpu.VMEM((DCPT, CH), jnp.int32),
            pltpu.VMEM((DCPT, CH), jnp.int32),
            pltpu.VMEM((CH,), jnp.float32),
            pltpu.VMEM((DRPT,), jnp.float32),
        ],
    )(pkall)


def _spmm_body(ycat, pkall, part, acc, pkchunk, srcb, dstb, rows,
               g0, g1, g2, i0, i1, i2):
    c = lax.axis_index("c")
    s = lax.axis_index("s")
    gsem = (g0, g1, g2)
    isem = (i0, i1, i2)

    def _idx_load(j, b):
        pltpu.async_copy(pkall.at[c, s, j], pkchunk.at[b], isem[b])

    def _idx_wait(b):
        pltpu.make_async_copy(
            pkall.at[c, s, 0], pkchunk.at[b], isem[b]).wait()

    def _unpack(b):
        for q in range(CH // 16):
            v = pkchunk[b, pl.ds(q * 16, 16)]
            srcb[b, pl.ds(q * 16, 16)] = lax.bitwise_and(v, 32767)
            dstb[b, pl.ds(q * 16, 16)] = lax.shift_right_logical(v, 15)

    def _gather(b):
        pltpu.async_copy(ycat.at[srcb.at[b]], rows.at[b], gsem[b])

    def _gather_wait(b):
        pltpu.make_async_copy(
            ycat.at[pl.ds(0, CH)], rows.at[b], gsem[b]).wait()

    def _scatter(b):
        pltpu.sync_copy(rows.at[b], acc.at[dstb.at[b]], add=True)

    for b in range(3):
        _idx_load(b, b)

    def _zero(i, t):
        for j in range(H // 16):
            rows[0, i, pl.ds(j * 16, 16)] = jnp.zeros((16,), jnp.float32)
        return t

    lax.fori_loop(0, CH, _zero, 0)
    for k in range(RPT // CH):
        pltpu.sync_copy(rows.at[0], acc.at[pl.ds(s * RPT + k * CH, CH)])
    for b in range(3):
        _idx_wait(b)
        _unpack(b)
        _gather(b)
        _idx_load(b + 3, b)
    plsc.subcore_barrier()

    def _step(t, u):
        for b in range(3):
            _gather_wait(b)
            _scatter(b)
            _idx_wait(b)
            _unpack(b)
            _gather(b)
            _idx_load(3 * t + b + 6, b)
        return u

    lax.fori_loop(0, (CPT - 7) // 3, _step, 0)
    for b in range(3):
        _gather_wait(b)
        _scatter(b)
        _idx_wait(b)
        _unpack(b)
        _gather(b)
    _idx_load(CPT - 1, 0)
    _gather_wait(0)
    _scatter(0)
    _idx_wait(0)
    _unpack(0)
    _gather(0)
    _gather_wait(1)
    _scatter(1)
    _gather_wait(2)
    _scatter(2)
    _gather_wait(0)
    _scatter(0)
    plsc.subcore_barrier()
    pltpu.sync_copy(acc.at[pl.ds(s * RPT, RPT)],
                    part.at[c, pl.ds(s * RPT, RPT)])


def _spmm(ycat, pkall):
    return pl.kernel(
        _spmm_body,
        out_type=jax.ShapeDtypeStruct((NC, NPAD, H), jnp.float32),
        mesh=_mesh(),
        scratch_types=[
            pltpu.VMEM_SHARED((NPAD, H), jnp.float32),
            pltpu.VMEM((3, CH), jnp.int32),
            pltpu.VMEM((3, CH), jnp.int32),
            pltpu.VMEM((3, CH), jnp.int32),
            pltpu.VMEM((3, CH, H), jnp.float32),
            pltpu.SemaphoreType.DMA,
            pltpu.SemaphoreType.DMA,
            pltpu.SemaphoreType.DMA,
            pltpu.SemaphoreType.DMA,
            pltpu.SemaphoreType.DMA,
            pltpu.SemaphoreType.DMA,
        ],
    )(ycat, pkall)


def _mm_scale_body(xo_ref, xc_ref, deg_ref, w_ref, o_ref):
    d = deg_ref[...]
    w = w_ref[...]
    yo = jnp.dot(xo_ref[...], w[0],
                 preferred_element_type=jnp.float32) * lax.rsqrt(d[0] + 1.0)
    yc = jnp.dot(xc_ref[...], w[1],
                 preferred_element_type=jnp.float32) * lax.rsqrt(d[1] + 1.0)
    o_ref[...] = jnp.stack([yo, yc])


def _mm_scale(x_o, x_c, degs, ws):
    return pl.pallas_call(
        _mm_scale_body,
        grid=(NBLK,),
        in_specs=[
            pl.BlockSpec((BLK, D), lambda i: (i, 0)),
            pl.BlockSpec((BLK, D), lambda i: (i, 0)),
            pl.BlockSpec((NC, BLK, 1), lambda i: (0, i, 0)),
            pl.BlockSpec((NC, D, H), lambda i: (0, 0, 0)),
        ],
        out_specs=pl.BlockSpec((NC, BLK, H), lambda i: (0, i, 0)),
        out_shape=jax.ShapeDtypeStruct((NC, N, H), jnp.float32),
    )(x_o, x_c, degs, ws)


def _combine_mm_body(p_ref, y_ref, deg_ref, b_ref, w_ref, o_ref):
    dis = lax.rsqrt(deg_ref[...][0] + 1.0)
    h = dis * (p_ref[...][0] + y_ref[...][0]) + b_ref[...][0]
    h = jnp.maximum(h, 0.0)
    o_ref[...] = (jnp.dot(
        h, w_ref[...][0], preferred_element_type=jnp.float32) * dis)[None]


def _combine_mm(part, y, degs, bs, ws):
    return pl.pallas_call(
        _combine_mm_body,
        grid=(NC, NBLK),
        in_specs=[
            pl.BlockSpec((1, BLK, H), lambda b, i: (b, i, 0)),
            pl.BlockSpec((1, BLK, H), lambda b, i: (b, i, 0)),
            pl.BlockSpec((1, BLK, 1), lambda b, i: (b, i, 0)),
            pl.BlockSpec((1, 1, H), lambda b, i: (b, 0, 0)),
            pl.BlockSpec((1, H, H), lambda b, i: (b, 0, 0)),
        ],
        out_specs=pl.BlockSpec((1, BLK, H), lambda b, i: (b, i, 0)),
        out_shape=jax.ShapeDtypeStruct((NC, N, H), jnp.float32),
    )(part, y, degs, bs, ws)


def _final_body(po_ref, yo_ref, dego_ref, bo_ref, pc_ref, yc_ref, degc_ref,
                bc_ref, batch_ref, wl_ref, bl_ref,
                xo_ref, xc_ref, hout_ref, pool_acc):
    i = pl.program_id(0)
    diso = lax.rsqrt(dego_ref[...][0] + 1.0)
    xo_ref[...] = diso * (po_ref[...][0] + yo_ref[...][0]) + bo_ref[...]
    disc = lax.rsqrt(degc_ref[...][0] + 1.0)
    xc = disc * (pc_ref[...][0] + yc_ref[...][0]) + bc_ref[...]
    xc_ref[...] = xc
    gids = lax.broadcasted_iota(jnp.int32, (BLK, G), 1)
    onehot = jnp.where(batch_ref[...] == gids, 1.0, 0.0)
    contrib = lax.dot_general(
        onehot, xc, (((0,), (0,)), ((), ())),
        preferred_element_type=jnp.float32)

    @pl.when(i == 0)
    def _():
        pool_acc[...] = contrib

    @pl.when(i > 0)
    def _():
        pool_acc[...] += contrib

    @pl.when(i == NBLK - 1)
    def _():
        hout_ref[...] = jnp.dot(
            pool_acc[...], wl_ref[...],
            preferred_element_type=jnp.float32) + bl_ref[...]


def _final(p2, y2, degs, bo, bc, batch, wl, bl):
    return pl.pallas_call(
        _final_body,
        grid=(NBLK,),
        in_specs=[
            pl.BlockSpec((1, BLK, H), lambda i: (0, i, 0)),
            pl.BlockSpec((1, BLK, H), lambda i: (0, i, 0)),
            pl.BlockSpec((1, BLK, 1), lambda i: (0, i, 0)),
            pl.BlockSpec((1, H), lambda i: (0, 0)),
            pl.BlockSpec((1, BLK, H), lambda i: (1, i, 0)),
            pl.BlockSpec((1, BLK, H), lambda i: (1, i, 0)),
            pl.BlockSpec((1, BLK, 1), lambda i: (1, i, 0)),
            pl.BlockSpec((1, H), lambda i: (0, 0)),
            pl.BlockSpec((BLK, 1), lambda i: (i, 0)),
            pl.BlockSpec((H, H), lambda i: (0, 0)),
            pl.BlockSpec((1, H), lambda i: (0, 0)),
        ],
        out_specs=[
            pl.BlockSpec((BLK, H), lambda i: (i, 0)),
            pl.BlockSpec((BLK, H), lambda i: (i, 0)),
            pl.BlockSpec((G, H), lambda i: (0, 0)),
        ],
        out_shape=[
            jax.ShapeDtypeStruct((N, H), jnp.float32),
            jax.ShapeDtypeStruct((N, H), jnp.float32),
            jax.ShapeDtypeStruct((G, H), jnp.float32),
        ],
        scratch_shapes=[pltpu.VMEM((G, H), jnp.float32)],
    )(p2, y2, degs, bo, p2, y2, degs, bc, batch, wl, bl)


def _prep_edges(edge_index, branch):
    src = edge_index[0].astype(jnp.int32) + branch * N
    dst = edge_index[1].astype(jnp.int32)
    return (dst * 32768 + src).reshape(NS, CPT, CH)


def kernel(x_o, x_c, edge_index_o, edge_index_c, batch_o,
           W1o, b1o, W2o, b2o, W1c, b1c, W2c, b2c, Wl1, bl1):
    pkall = jnp.stack(
        [_prep_edges(edge_index_o, 0), _prep_edges(edge_index_c, 1)])
    deg2 = _degrees(pkall)
    degs = deg2.reshape(NC, NPAD, 1)
    batch_p = batch_o.astype(jnp.int32).reshape(N, 1)

    y1 = _mm_scale(x_o.astype(jnp.float32), x_c.astype(jnp.float32),
                   degs, jnp.stack([W1o, W1c]))
    p1 = _spmm(y1.reshape(NC * N, H), pkall)
    y2 = _combine_mm(p1, y1, degs,
                     jnp.stack([b1o.reshape(1, H), b1c.reshape(1, H)]),
                     jnp.stack([W2o, W2c]))
    p2 = _spmm(y2.reshape(NC * N, H), pkall)
    xo2, xc2, h_out = _final(
        p2, y2, degs, b2o.reshape(1, H), b2c.reshape(1, H),
        batch_p, Wl1, bl1.reshape(1, H))
    return (h_out, xo2, xc2)
